# baseline (speedup 1.0000x reference)
import jax
import jax.numpy as jnp
from jax import lax
from jax.experimental import pallas as pl
from jax.experimental.pallas import tpu as pltpu
from jax.experimental.pallas import tpu_sc as plsc

_NUM_CORES = 2
_NUM_SUBCORES = 16
_NUM_WORKERS = _NUM_CORES * _NUM_SUBCORES

_STRIP = 16384
_PACK8 = 8
_SC_ROWS = 507904
_CH = 512


def _mesh():
    return plsc.VectorSubcoreMesh(
        core_axis_name="c", subcore_axis_name="s",
        num_cores=_NUM_CORES, num_subcores=_NUM_SUBCORES)


def _wid():
    return lax.axis_index("s") * _NUM_CORES + lax.axis_index("c")


def _score_body(c_ref, tbl_ref, out_ref):
    i = pl.program_id(0)
    x = tbl_ref[...]
    s = lax.dot_general(c_ref[...], x, (((1,), (1,)), ((), ())),
                        preferred_element_type=jnp.float32)
    out_ref[pl.ds(lax.rem(i, _PACK8), 1), :] = s


def _tc_scores(object_table, v1):
    v, d = object_table.shape
    c2d = lax.slice(object_table, (0, 0), (1, d))
    nblk = -(-v1 // _STRIP)
    nout = -(-nblk // _PACK8)
    out = pl.pallas_call(
        _score_body,
        grid=(nblk,),
        in_specs=[
            pl.BlockSpec((1, d), lambda i: (0, 0)),
            pl.BlockSpec((_STRIP, d), lambda i: (i, 0)),
        ],
        out_specs=pl.BlockSpec((_PACK8, _STRIP), lambda i: (i // _PACK8, 0)),
        out_shape=jax.ShapeDtypeStruct((nout * _PACK8, _STRIP), jnp.float32),
    )(c2d, object_table)
    return out.reshape(nout * _PACK8 * _STRIP)


def _sc_score_body(d, row0, per_w, tbl_hbm, out_hbm, cbuf, rows_v, out_v, sem):
    base = row0 + _wid() * per_w
    pltpu.sync_copy(tbl_hbm.at[pl.ds(0, 1), :], cbuf)
    cvecs = [cbuf[0, pl.ds(q * 16, 16)] for q in range(d // 16)]
    ck = [cvecs[k // 16][k % 16] for k in range(d)]
    lanes = lax.iota(jnp.int32, 16)

    def chunk(j, carry):
        pltpu.async_copy(tbl_hbm.at[pl.ds(base + j * _CH, _CH), :],
                         rows_v, sem).wait()

        def group(g, c2):
            rowid = g * 16 + lanes
            acc = jnp.zeros((16,), jnp.float32)
            for k in range(d):
                col = plsc.load_gather(
                    rows_v, [rowid, jnp.full((16,), k, jnp.int32)])
                acc = acc + ck[k] * col
            out_v[pl.ds(j * _CH + g * 16, 16)] = acc
            return c2

        lax.fori_loop(0, _CH // 16, group, 0)
        return carry

    lax.fori_loop(0, per_w // _CH, chunk, 0)
    pltpu.sync_copy(out_v, out_hbm.at[pl.ds(_wid() * per_w, per_w)])


def _sc_scores(object_table, row0):
    v, d = object_table.shape
    per_w = (v - row0) // _NUM_WORKERS

    def body(tbl_hbm, out_hbm, cbuf, rows_v, out_v, sem):
        _sc_score_body(d, row0, per_w, tbl_hbm, out_hbm, cbuf, rows_v, out_v, sem)

    f = pl.kernel(
        body,
        mesh=_mesh(),
        compiler_params=pltpu.CompilerParams(needs_layout_passes=False),
        out_type=jax.ShapeDtypeStruct((v - row0,), jnp.float32),
        scratch_types=[
            pltpu.VMEM((1, d), jnp.float32),
            pltpu.VMEM((_CH, d), jnp.float32),
            pltpu.VMEM((per_w,), jnp.float32),
            pltpu.SemaphoreType.DMA,
        ],
    )
    return f(object_table)


def _gather_body(per_w, scores_hbm, idx_hbm, out_hbm, idx_v, out_v, sem):
    base = _wid() * per_w
    pltpu.sync_copy(idx_hbm.at[pl.ds(base, per_w)], idx_v)
    pltpu.async_copy(scores_hbm.at[idx_v], out_v, sem).wait()
    pltpu.sync_copy(out_v, out_hbm.at[pl.ds(base, per_w)])


def _gather_scores(scores, idx_flat):
    n = idx_flat.shape[0]
    per_w = n // _NUM_WORKERS

    def body(scores_hbm, idx_hbm, out_hbm, idx_v, out_v, sem):
        _gather_body(per_w, scores_hbm, idx_hbm, out_hbm, idx_v, out_v, sem)

    f = pl.kernel(
        body,
        mesh=_mesh(),
        out_type=jax.ShapeDtypeStruct((n,), jnp.float32),
        scratch_types=[
            pltpu.VMEM((per_w,), jnp.int32),
            pltpu.VMEM((per_w,), jnp.float32),
            pltpu.SemaphoreType.DMA,
        ],
    )
    return f(scores, idx_flat)


def kernel(indices, object_table):
    b, l = indices.shape
    v, _ = object_table.shape
    v1 = v - _SC_ROWS
    scores_lo = _tc_scores(object_table, v1)
    scores_hi = _sc_scores(object_table, v1)
    scores = jnp.concatenate([scores_lo[:v1], scores_hi])
    out = _gather_scores(scores, indices.reshape(-1))
    return out.reshape(b, l)


# speedup vs baseline: 1.2462x; 1.2462x over previous
import jax
import jax.numpy as jnp
from jax import lax
from jax.experimental import pallas as pl
from jax.experimental.pallas import tpu as pltpu
from jax.experimental.pallas import tpu_sc as plsc

_NUM_CORES = 2
_NUM_SUBCORES = 16
_NUM_WORKERS = _NUM_CORES * _NUM_SUBCORES

_STRIP = 16384
_PACK8 = 8
_SC_ROWS = 327680
_CH = 512


def _mesh():
    return plsc.VectorSubcoreMesh(
        core_axis_name="c", subcore_axis_name="s",
        num_cores=_NUM_CORES, num_subcores=_NUM_SUBCORES)


def _wid():
    return lax.axis_index("s") * _NUM_CORES + lax.axis_index("c")


def _score_body(c_ref, tbl_ref, out_ref):
    i = pl.program_id(0)
    x = tbl_ref[...]
    s = lax.dot_general(c_ref[...], x, (((1,), (1,)), ((), ())),
                        preferred_element_type=jnp.float32)
    out_ref[pl.ds(lax.rem(i, _PACK8), 1), :] = s


def _tc_scores(object_table, v1):
    v, d = object_table.shape
    c2d = lax.slice(object_table, (0, 0), (1, d))
    nblk = -(-v1 // _STRIP)
    nout = -(-nblk // _PACK8)
    out = pl.pallas_call(
        _score_body,
        grid=(nblk,),
        in_specs=[
            pl.BlockSpec((1, d), lambda i: (0, 0)),
            pl.BlockSpec((_STRIP, d), lambda i: (i, 0)),
        ],
        out_specs=pl.BlockSpec((_PACK8, _STRIP), lambda i: (i // _PACK8, 0)),
        out_shape=jax.ShapeDtypeStruct((nout * _PACK8, _STRIP), jnp.float32),
    )(c2d, object_table)
    return out.reshape(nout * _PACK8 * _STRIP)


def _sc_score_body(d, row0, per_w, tbl_hbm, out_hbm, cbuf, rows_v, out_v, sem):
    base = row0 + _wid() * per_w
    pltpu.sync_copy(tbl_hbm.at[pl.ds(0, 1), :], cbuf)
    cvecs = [cbuf[0, pl.ds(q * 16, 16)] for q in range(d // 16)]
    ck = [cvecs[k // 16][k % 16] for k in range(d)]
    lanes = lax.iota(jnp.int32, 16)

    def chunk(j, carry):
        pltpu.async_copy(tbl_hbm.at[pl.ds(base + j * _CH, _CH), :],
                         rows_v, sem).wait()

        def group(g, c2):
            rowid = g * 16 + lanes
            acc = jnp.zeros((16,), jnp.float32)
            for k in range(d):
                col = plsc.load_gather(
                    rows_v, [rowid, jnp.full((16,), k, jnp.int32)])
                acc = acc + ck[k] * col
            out_v[pl.ds(j * _CH + g * 16, 16)] = acc
            return c2

        lax.fori_loop(0, _CH // 16, group, 0)
        return carry

    lax.fori_loop(0, per_w // _CH, chunk, 0)
    pltpu.sync_copy(out_v, out_hbm.at[pl.ds(_wid() * per_w, per_w)])


def _sc_scores(object_table, row0):
    v, d = object_table.shape
    per_w = (v - row0) // _NUM_WORKERS

    def body(tbl_hbm, out_hbm, cbuf, rows_v, out_v, sem):
        _sc_score_body(d, row0, per_w, tbl_hbm, out_hbm, cbuf, rows_v, out_v, sem)

    f = pl.kernel(
        body,
        mesh=_mesh(),
        compiler_params=pltpu.CompilerParams(needs_layout_passes=False),
        out_type=jax.ShapeDtypeStruct((v - row0,), jnp.float32),
        scratch_types=[
            pltpu.VMEM((1, d), jnp.float32),
            pltpu.VMEM((_CH, d), jnp.float32),
            pltpu.VMEM((per_w,), jnp.float32),
            pltpu.SemaphoreType.DMA,
        ],
    )
    return f(object_table)


def _gather_body(per_w, scores_hbm, idx_hbm, out_hbm, idx_v, out_v, sem):
    base = _wid() * per_w
    pltpu.sync_copy(idx_hbm.at[pl.ds(base, per_w)], idx_v)
    pltpu.async_copy(scores_hbm.at[idx_v], out_v, sem).wait()
    pltpu.sync_copy(out_v, out_hbm.at[pl.ds(base, per_w)])


def _gather_scores(scores, idx_flat):
    n = idx_flat.shape[0]
    per_w = n // _NUM_WORKERS

    def body(scores_hbm, idx_hbm, out_hbm, idx_v, out_v, sem):
        _gather_body(per_w, scores_hbm, idx_hbm, out_hbm, idx_v, out_v, sem)

    f = pl.kernel(
        body,
        mesh=_mesh(),
        out_type=jax.ShapeDtypeStruct((n,), jnp.float32),
        scratch_types=[
            pltpu.VMEM((per_w,), jnp.int32),
            pltpu.VMEM((per_w,), jnp.float32),
            pltpu.SemaphoreType.DMA,
        ],
    )
    return f(scores, idx_flat)


def kernel(indices, object_table):
    b, l = indices.shape
    v, _ = object_table.shape
    v1 = v - _SC_ROWS
    scores_lo = _tc_scores(object_table, v1)
    scores_hi = _sc_scores(object_table, v1)
    scores = jnp.concatenate([scores_lo[:v1], scores_hi])
    out = _gather_scores(scores, indices.reshape(-1))
    return out.reshape(b, l)


# speedup vs baseline: 1.6726x; 1.3421x over previous
import jax
import jax.numpy as jnp
from jax import lax
from jax.experimental import pallas as pl
from jax.experimental.pallas import tpu as pltpu
from jax.experimental.pallas import tpu_sc as plsc

_NUM_CORES = 2
_NUM_SUBCORES = 16
_NUM_WORKERS = _NUM_CORES * _NUM_SUBCORES

_STRIP = 32768
_PACK8 = 8


def _score_body(c_ref, tbl_ref, out_ref):
    i = pl.program_id(0)
    x = tbl_ref[...]
    s = lax.dot_general(c_ref[...], x, (((1,), (1,)), ((), ())),
                        preferred_element_type=jnp.float32)
    out_ref[pl.ds(lax.rem(i, _PACK8), 1), :] = s


def _compute_scores(object_table):
    v, d = object_table.shape
    c2d = lax.slice(object_table, (0, 0), (1, d))
    nblk = -(-v // _STRIP)
    nout = -(-nblk // _PACK8)
    out = pl.pallas_call(
        _score_body,
        grid=(nblk,),
        in_specs=[
            pl.BlockSpec((1, d), lambda i: (0, 0)),
            pl.BlockSpec((_STRIP, d), lambda i: (i, 0)),
        ],
        out_specs=pl.BlockSpec((_PACK8, _STRIP), lambda i: (i // _PACK8, 0)),
        out_shape=jax.ShapeDtypeStruct((nout * _PACK8, _STRIP), jnp.float32),
    )(c2d, object_table)
    return out.reshape(nout * _PACK8 * _STRIP)


def _gather_body(per_w, scores_hbm, idx_hbm, out_hbm, idx_v, out_v, sem):
    wid = lax.axis_index("s") * _NUM_CORES + lax.axis_index("c")
    base = wid * per_w
    pltpu.sync_copy(idx_hbm.at[pl.ds(base, per_w)], idx_v)
    pltpu.async_copy(scores_hbm.at[idx_v], out_v, sem).wait()
    pltpu.sync_copy(out_v, out_hbm.at[pl.ds(base, per_w)])


def _gather_scores(scores, idx_flat):
    n = idx_flat.shape[0]
    per_w = n // _NUM_WORKERS
    mesh = plsc.VectorSubcoreMesh(
        core_axis_name="c", subcore_axis_name="s",
        num_cores=_NUM_CORES, num_subcores=_NUM_SUBCORES)

    def body(scores_hbm, idx_hbm, out_hbm, idx_v, out_v, sem):
        _gather_body(per_w, scores_hbm, idx_hbm, out_hbm, idx_v, out_v, sem)

    f = pl.kernel(
        body,
        mesh=mesh,
        out_type=jax.ShapeDtypeStruct((n,), jnp.float32),
        scratch_types=[
            pltpu.VMEM((per_w,), jnp.int32),
            pltpu.VMEM((per_w,), jnp.float32),
            pltpu.SemaphoreType.DMA,
        ],
    )
    return f(scores, idx_flat)


def kernel(indices, object_table):
    b, l = indices.shape
    scores = _compute_scores(object_table)
    out = _gather_scores(scores, indices.reshape(-1))
    return out.reshape(b, l)
